# baseline (speedup 1.0000x reference)
import dataclasses

import numpy as np

import jax
import jax.numpy as jnp
from jax import lax
from jax.experimental import pallas as pl
from jax.experimental.pallas import tpu as pltpu
from jax.experimental.pallas import tpu_sc as plsc

B = 32
DIM = 96
N = 1024
LTXT = 77
K = 512

_I32_MIN = np.int32(-(2**31))


def _fused_scores_tc_kernel(img_ref, txt_ref, wf_ref, bf_ref,
                            g_ref, be_ref, ws_ref, fused_ref, sw_ref,
                            thr_ref, quota_ref, wacc_ref):
    img = img_ref[0]
    txt = txt_ref[0]

    x = jnp.transpose(img)
    tmean = jnp.mean(txt, axis=0, keepdims=True)
    tb = jnp.broadcast_to(tmean, (N, DIM))
    z = jnp.zeros((N, 128 - DIM), jnp.float32)
    cat = jnp.concatenate([x, z, tb, z], axis=1)
    fused = jnp.dot(cat, wf_ref[...]) + bf_ref[...]

    mu = jnp.mean(fused, axis=1, keepdims=True)
    d = fused - mu
    var = jnp.mean(d * d, axis=1, keepdims=True)
    normed = d / jnp.sqrt(var + 1e-5) * g_ref[...] + be_ref[...]

    s_col = jnp.dot(normed, ws_ref[...])
    s_row = jnp.reshape(s_col, (1, N))

    si = lax.bitcast_convert_type(s_row, jnp.int32)
    w = jnp.where(si >= 0, si, jnp.bitwise_xor(jnp.bitwise_not(si), _I32_MIN))

    sw_ref[0] = w
    fused_ref[0] = jnp.concatenate([fused, z], axis=1)

    i = pl.program_id(0)
    wacc_ref[pl.ds(i, 1), :] = w

    @pl.when(i == B - 1)
    def _():
        wa = wacc_ref[...]
        cur = jnp.full((B, 1), _I32_MIN, jnp.int32)
        cnt0 = jnp.sum((wa >= 0).astype(jnp.int32), axis=1, keepdims=True)
        cur = jnp.where(cnt0 >= K, jnp.zeros((B, 1), jnp.int32), cur)

        def body(t, cur):
            c = cur + (jnp.int32(1) << (30 - t))
            cnt = jnp.sum((wa >= c).astype(jnp.int32), axis=1, keepdims=True)
            return jnp.where(cnt >= K, c, cur)

        cur = lax.fori_loop(0, 31, body, cur)
        ngt = jnp.sum((wa > cur).astype(jnp.int32), axis=1, keepdims=True)
        thr_ref[...] = jnp.broadcast_to(cur, (B, 16))
        quota_ref[...] = jnp.broadcast_to(K - ngt, (B, 16))


def _select_gather_sc_kernel(off, sw_hbm, thr_hbm, quota_hbm, fused_hbm,
                             out_hbm, sv, tv, qv, idxv, rows, sem):
    wid = lax.axis_index("s") * 2 + lax.axis_index("c")
    b = off + wid // 2
    half = wid % 2

    pltpu.sync_copy(sw_hbm.at[b], sv)
    pltpu.sync_copy(thr_hbm.at[b], tv)
    pltpu.sync_copy(quota_hbm.at[b], qv)

    tvec = tv[...]
    qvec = qv[...]

    def chunk(c, carry):
        npos, neq = carry
        v = sv[pl.ds(c * 16, 16)]
        gt = v > tvec
        eq = v == tvec
        eqi = eq.astype(jnp.int32)
        eq_rank = plsc.cumsum(eqi) - eqi + neq
        sel = jnp.logical_or(gt, jnp.logical_and(eq, eq_rank < qvec))
        seli = sel.astype(jnp.int32)
        pos = plsc.cumsum(seli) - 1 + npos
        gidx = lax.iota(jnp.int32, 16) + (c * 16 + b * N)
        plsc.store_scatter(idxv, [pos], gidx, mask=sel)
        return npos + jnp.sum(seli), neq + jnp.sum(eqi)

    lax.fori_loop(0, N // 16, chunk, (jnp.int32(0), jnp.int32(0)))

    ob = (b - off) * K
    k0 = half * 256
    cp0 = pltpu.async_copy(
        fused_hbm.at[idxv.at[pl.ds(k0, 128)]], rows.at[0], sem)
    cp1 = pltpu.async_copy(
        fused_hbm.at[idxv.at[pl.ds(k0 + 128, 128)]], rows.at[1], sem)
    cp0.wait()
    pltpu.sync_copy(rows.at[0], out_hbm.at[pl.ds(ob + k0, 128)])
    cp1.wait()
    pltpu.sync_copy(rows.at[1], out_hbm.at[pl.ds(ob + k0 + 128, 128)])


def kernel(img_tokens, text_tokens, W_fusion, b_fusion, gamma, beta,
           W_score, b_score):
    imgf = img_tokens.reshape(B, DIM, N)
    bf = b_fusion.reshape(1, DIM)
    g_row = gamma.reshape(1, DIM)
    be_row = beta.reshape(1, DIM)
    zw = jnp.zeros((128 - DIM, DIM), jnp.float32)
    wf_pad = jnp.concatenate([W_fusion[:DIM], zw, W_fusion[DIM:], zw], axis=0)

    fused, sw, thr, quota = pl.pallas_call(
        _fused_scores_tc_kernel,
        grid=(B,),
        in_specs=[
            pl.BlockSpec((1, DIM, N), lambda i: (i, 0, 0)),
            pl.BlockSpec((1, LTXT, DIM), lambda i: (i, 0, 0)),
            pl.BlockSpec((256, DIM), lambda i: (0, 0)),
            pl.BlockSpec((1, DIM), lambda i: (0, 0)),
            pl.BlockSpec((1, DIM), lambda i: (0, 0)),
            pl.BlockSpec((1, DIM), lambda i: (0, 0)),
            pl.BlockSpec((DIM, 1), lambda i: (0, 0)),
        ],
        out_specs=[
            pl.BlockSpec((1, N, 128), lambda i: (i, 0, 0)),
            pl.BlockSpec((1, 1, N), lambda i: (i, 0, 0)),
            pl.BlockSpec((B, 16), lambda i: (0, 0)),
            pl.BlockSpec((B, 16), lambda i: (0, 0)),
        ],
        out_shape=[
            jax.ShapeDtypeStruct((B, N, 128), jnp.float32),
            jax.ShapeDtypeStruct((B, 1, N), jnp.int32),
            jax.ShapeDtypeStruct((B, 16), jnp.int32),
            jax.ShapeDtypeStruct((B, 16), jnp.int32),
        ],
        scratch_shapes=[pltpu.VMEM((B, N), jnp.int32)],
    )(imgf, text_tokens, wf_pad, bf, g_row, be_row, W_score)

    sw2 = sw.reshape(B, N)
    fused_flat = fused.reshape(B * N, 128)
    outs = []
    for h in range(2):
        of = _run_sc(h * (B // 2), sw2, thr, quota, fused_flat)
        outs.append(of.reshape(B // 2, K, 128)[:, :, :DIM])
    return jnp.concatenate(outs, axis=0)


def _run_sc(off, sw2, thr, quota, fused_flat):
    cp = pltpu.CompilerParams()
    if "needs_layout_passes" in pltpu.CompilerParams.__dataclass_fields__:
        cp = dataclasses.replace(cp, needs_layout_passes=False)

    import functools as _ft
    sc_kernel = pl.kernel(
        _ft.partial(_select_gather_sc_kernel, off),
        out_type=jax.ShapeDtypeStruct((B // 2 * K, 128), jnp.float32),
        mesh=plsc.VectorSubcoreMesh(core_axis_name="c", subcore_axis_name="s"),
        compiler_params=cp,
        scratch_types=[
            pltpu.VMEM((N,), jnp.int32),
            pltpu.VMEM((16,), jnp.int32),
            pltpu.VMEM((16,), jnp.int32),
            pltpu.VMEM((K,), jnp.int32),
            pltpu.VMEM((2, 128, 128), jnp.float32),
            pltpu.SemaphoreType.DMA,
        ],
    )
    return sc_kernel(sw2, thr, quota, fused_flat)


# speedup vs baseline: 1.2868x; 1.2868x over previous
import dataclasses

import numpy as np

import jax
import jax.numpy as jnp
from jax import lax
from jax.experimental import pallas as pl
from jax.experimental.pallas import tpu as pltpu
from jax.experimental.pallas import tpu_sc as plsc

B = 32
DIM = 96
N = 1024
LTXT = 77
K = 512

_I32_MIN = np.int32(-(2**31))


def _fused_scores_tc_kernel(img_ref, txt_ref, wf_ref, bf_ref,
                            g_ref, be_ref, ws_ref, fused_ref, sw_ref,
                            thr_ref, quota_ref, wacc_ref):
    x = img_ref[0]
    txt = txt_ref[0]
    tmean = jnp.mean(txt, axis=0, keepdims=True)
    tb = jnp.broadcast_to(tmean, (N, DIM))
    z = jnp.zeros((N, 128 - DIM), jnp.float32)
    cat = jnp.concatenate([x, z, tb, z], axis=1)
    fused = jnp.dot(cat, wf_ref[...]) + bf_ref[...]

    mu = jnp.mean(fused, axis=1, keepdims=True)
    d = fused - mu
    var = jnp.mean(d * d, axis=1, keepdims=True)
    normed = d / jnp.sqrt(var + 1e-5) * g_ref[...] + be_ref[...]

    s_col = jnp.dot(normed, ws_ref[...])
    s_row = jnp.reshape(s_col, (1, N))

    si = lax.bitcast_convert_type(s_row, jnp.int32)
    w = jnp.where(si >= 0, si, jnp.bitwise_xor(jnp.bitwise_not(si), _I32_MIN))

    sw_ref[0] = w
    fused_ref[0] = jnp.concatenate([fused, z], axis=1)

    i = pl.program_id(0)
    wacc_ref[pl.ds(i, 1), :] = w

    @pl.when(i == B - 1)
    def _():
        wa = wacc_ref[...]
        cur = jnp.full((B, 1), _I32_MIN, jnp.int32)
        cnt0 = jnp.sum((wa >= 0).astype(jnp.int32), axis=1, keepdims=True)
        cur = jnp.where(cnt0 >= K, jnp.zeros((B, 1), jnp.int32), cur)

        def body(t, cur):
            c = cur + (jnp.int32(1) << (30 - t))
            cnt = jnp.sum((wa >= c).astype(jnp.int32), axis=1, keepdims=True)
            return jnp.where(cnt >= K, c, cur)

        cur = lax.fori_loop(0, 31, body, cur)
        ngt = jnp.sum((wa > cur).astype(jnp.int32), axis=1, keepdims=True)
        thr_ref[...] = jnp.broadcast_to(cur, (B, 16))
        quota_ref[...] = jnp.broadcast_to(K - ngt, (B, 16))


def _select_gather_sc_kernel(sw_hbm, thr_hbm, quota_hbm, fused_hbm,
                             out_hbm, sv, tv, qv, idxv, rows, sem):
    wid = lax.axis_index("s") * 2 + lax.axis_index("c")
    b = wid

    pltpu.sync_copy(sw_hbm.at[b], sv)
    pltpu.sync_copy(thr_hbm.at[b], tv)
    pltpu.sync_copy(quota_hbm.at[b], qv)

    tvec = tv[...]
    qvec = qv[...]

    def chunk(c, carry):
        npos, neq = carry
        v = sv[pl.ds(c * 16, 16)]
        gt = v > tvec
        eq = v == tvec
        eqi = eq.astype(jnp.int32)
        eq_rank = plsc.cumsum(eqi) - eqi + neq
        sel = jnp.logical_or(gt, jnp.logical_and(eq, eq_rank < qvec))
        seli = sel.astype(jnp.int32)
        pos = plsc.cumsum(seli) - 1 + npos
        gidx = lax.iota(jnp.int32, 16) + (c * 16 + b * N)
        plsc.store_scatter(idxv, [pos], gidx, mask=sel)
        return npos + jnp.sum(seli), neq + jnp.sum(eqi)

    lax.fori_loop(0, N // 16, chunk, (jnp.int32(0), jnp.int32(0)))

    cp0 = pltpu.async_copy(
        fused_hbm.at[idxv.at[pl.ds(0, 128)]], rows.at[0], sem)
    cp1 = pltpu.async_copy(
        fused_hbm.at[idxv.at[pl.ds(128, 128)]], rows.at[1], sem)
    cp0.wait()
    pltpu.sync_copy(rows.at[0], out_hbm.at[pl.ds(b * K, 128)])
    cp2 = pltpu.async_copy(
        fused_hbm.at[idxv.at[pl.ds(256, 128)]], rows.at[0], sem)
    cp1.wait()
    pltpu.sync_copy(rows.at[1], out_hbm.at[pl.ds(b * K + 128, 128)])
    cp3 = pltpu.async_copy(
        fused_hbm.at[idxv.at[pl.ds(384, 128)]], rows.at[1], sem)
    cp2.wait()
    pltpu.sync_copy(rows.at[0], out_hbm.at[pl.ds(b * K + 256, 128)])
    cp3.wait()
    pltpu.sync_copy(rows.at[1], out_hbm.at[pl.ds(b * K + 384, 128)])


def kernel(img_tokens, text_tokens, W_fusion, b_fusion, gamma, beta,
           W_score, b_score):
    imgf = jnp.transpose(img_tokens, (0, 2, 3, 1)).reshape(B, N, DIM)
    bf = b_fusion.reshape(1, DIM)
    g_row = gamma.reshape(1, DIM)
    be_row = beta.reshape(1, DIM)
    zw = jnp.zeros((128 - DIM, DIM), jnp.float32)
    wf_pad = jnp.concatenate([W_fusion[:DIM], zw, W_fusion[DIM:], zw], axis=0)

    fused, sw, thr, quota = pl.pallas_call(
        _fused_scores_tc_kernel,
        grid=(B,),
        in_specs=[
            pl.BlockSpec((1, N, DIM), lambda i: (i, 0, 0)),
            pl.BlockSpec((1, LTXT, DIM), lambda i: (i, 0, 0)),
            pl.BlockSpec((256, DIM), lambda i: (0, 0)),
            pl.BlockSpec((1, DIM), lambda i: (0, 0)),
            pl.BlockSpec((1, DIM), lambda i: (0, 0)),
            pl.BlockSpec((1, DIM), lambda i: (0, 0)),
            pl.BlockSpec((DIM, 1), lambda i: (0, 0)),
        ],
        out_specs=[
            pl.BlockSpec((1, N, 128), lambda i: (i, 0, 0)),
            pl.BlockSpec((1, 1, N), lambda i: (i, 0, 0)),
            pl.BlockSpec((B, 16), lambda i: (0, 0)),
            pl.BlockSpec((B, 16), lambda i: (0, 0)),
        ],
        out_shape=[
            jax.ShapeDtypeStruct((B, N, 128), jnp.float32),
            jax.ShapeDtypeStruct((B, 1, N), jnp.int32),
            jax.ShapeDtypeStruct((B, 16), jnp.int32),
            jax.ShapeDtypeStruct((B, 16), jnp.int32),
        ],
        scratch_shapes=[pltpu.VMEM((B, N), jnp.int32)],
    )(imgf, text_tokens, wf_pad, bf, g_row, be_row, W_score)

    out_flat = _run_sc(sw.reshape(B, N), thr, quota, fused.reshape(B * N, 128))
    return out_flat.reshape(B, K, 128)[:, :, :DIM]


def _run_sc(sw2, thr, quota, fused_flat):
    cp = pltpu.CompilerParams()
    if "needs_layout_passes" in pltpu.CompilerParams.__dataclass_fields__:
        cp = dataclasses.replace(cp, needs_layout_passes=False)

    sc_kernel = pl.kernel(
        _select_gather_sc_kernel,
        out_type=jax.ShapeDtypeStruct((B * K, 128), jnp.float32),
        mesh=plsc.VectorSubcoreMesh(core_axis_name="c", subcore_axis_name="s"),
        compiler_params=cp,
        scratch_types=[
            pltpu.VMEM((N,), jnp.int32),
            pltpu.VMEM((16,), jnp.int32),
            pltpu.VMEM((16,), jnp.int32),
            pltpu.VMEM((K,), jnp.int32),
            pltpu.VMEM((2, 128, 128), jnp.float32),
            pltpu.SemaphoreType.DMA,
        ],
    )
    return sc_kernel(sw2, thr, quota, fused_flat)
